# baseline (speedup 1.0000x reference)
import functools

import jax
import jax.numpy as jnp
from jax import lax
from jax.experimental import pallas as pl
from jax.experimental.pallas import tpu as pltpu
from jax.experimental.pallas import tpu_sc as plsc

_L = 16
_NC = 2
_NS = 16
_NW = _NC * _NS
_P = 16

_S_SC = 3072
_S_BLK = 256


def _sc_part(x, pos, seq_lo, seq_hi):
    batch, seq_len, d_model = x.shape
    rows = seq_hi - seq_lo
    rows_per_w = rows // _NW
    steps = rows_per_w // _P
    ncol = d_model // _L

    mesh = plsc.VectorSubcoreMesh(core_axis_name="c", subcore_axis_name="s")

    @functools.partial(
        pl.kernel,
        mesh=mesh,
        out_type=jax.ShapeDtypeStruct((batch, rows, d_model), jnp.float32),
        scratch_types=(
            pltpu.VMEM((2, _P, d_model), jnp.float32),
            pltpu.VMEM((2, batch, _P, d_model), jnp.float32),
            pltpu.SemaphoreType.DMA,
            pltpu.SemaphoreType.DMA,
            pltpu.SemaphoreType.DMA,
            pltpu.SemaphoreType.DMA,
        ),
    )
    def sc_add(x_hbm, pos_hbm, out_hbm, pos_v, x_v, lsem0, lsem1, ssem0, ssem1):
        wid = lax.axis_index("s") * _NC + lax.axis_index("c")
        base = wid * rows_per_w
        lsems = (lsem0, lsem1)
        ssems = (ssem0, ssem1)

        def fire_loads(t, s):
            r0 = base + t * _P
            hs = [
                pltpu.async_copy(
                    pos_hbm.at[pl.ds(seq_lo + r0, _P)], pos_v.at[s], lsems[s]
                )
            ]
            for b in range(batch):
                hs.append(
                    pltpu.async_copy(
                        x_hbm.at[b, pl.ds(seq_lo + r0, _P)], x_v.at[s, b], lsems[s]
                    )
                )
            return hs

        def fire_stores(t, s):
            r0 = base + t * _P
            return [
                pltpu.async_copy(x_v.at[s, b], out_hbm.at[b, pl.ds(r0, _P)], ssems[s])
                for b in range(batch)
            ]

        def compute(s):
            def row_body(r, carry):
                for c in range(ncol):
                    pv = pos_v[s, r, pl.ds(c * _L, _L)]
                    for b in range(batch):
                        plsc.addupdate(x_v.at[s, b, r, pl.ds(c * _L, _L)], pv)
                return carry

            lax.fori_loop(0, _P, row_body, 0)

        pending_loads = {0: fire_loads(0, 0)}
        pending_stores = {}
        for t in range(steps):
            s = t & 1
            for h in pending_loads.pop(t):
                h.wait()
            if t + 1 < steps:
                if t - 1 in pending_stores:
                    for h in pending_stores.pop(t - 1):
                        h.wait()
                pending_loads[t + 1] = fire_loads(t + 1, s ^ 1)
            compute(s)
            pending_stores[t] = fire_stores(t, s)
        for ts in sorted(pending_stores):
            for h in pending_stores[ts]:
                h.wait()

    return sc_add(x, pos)


def _tc_body(x_ref, pos_ref, out_ref):
    out_ref[...] = x_ref[...] + pos_ref[...][None, :, :]


def _tc_part(x, pos, seq_lo, seq_hi):
    batch, seq_len, d_model = x.shape
    rows = seq_hi - seq_lo
    n_blocks = rows // _S_BLK
    blk_off = seq_lo // _S_BLK

    return pl.pallas_call(
        _tc_body,
        grid=(n_blocks,),
        in_specs=[
            pl.BlockSpec((batch, _S_BLK, d_model), lambda i: (0, i + blk_off, 0)),
            pl.BlockSpec((_S_BLK, d_model), lambda i: (i + blk_off, 0)),
        ],
        out_specs=pl.BlockSpec((batch, _S_BLK, d_model), lambda i: (0, i, 0)),
        out_shape=jax.ShapeDtypeStruct((batch, rows, d_model), x.dtype),
        compiler_params=pltpu.CompilerParams(
            dimension_semantics=("arbitrary",),
        ),
    )(x, pos)


def kernel(x, pos_embedding):
    batch, seq_len, d_model = x.shape
    pos = pos_embedding[:seq_len]
    sc_out = _sc_part(x, pos, 0, _S_SC)
    tc_out = _tc_part(x, pos, _S_SC, seq_len)
    return jnp.concatenate([sc_out, tc_out], axis=1)


# speedup vs baseline: 1.3107x; 1.3107x over previous
import functools

import jax
import jax.numpy as jnp
from jax import lax
from jax.experimental import pallas as pl
from jax.experimental.pallas import tpu as pltpu
from jax.experimental.pallas import tpu_sc as plsc

_L = 16
_NC = 2
_NS = 16
_NW = _NC * _NS
_P = 16


def kernel(x, pos_embedding):
    batch, seq_len, d_model = x.shape
    pos = pos_embedding[:seq_len]
    rows_per_w = seq_len // _NW
    steps = rows_per_w // _P
    ncol = d_model // _L

    mesh = plsc.VectorSubcoreMesh(core_axis_name="c", subcore_axis_name="s")

    @functools.partial(
        pl.kernel,
        mesh=mesh,
        out_type=jax.ShapeDtypeStruct((batch, seq_len, d_model), jnp.float32),
        scratch_types=(
            pltpu.VMEM((2, _P, d_model), jnp.float32),
            pltpu.VMEM((2, batch, _P, d_model), jnp.float32),
            pltpu.SemaphoreType.DMA,
            pltpu.SemaphoreType.DMA,
            pltpu.SemaphoreType.DMA,
            pltpu.SemaphoreType.DMA,
        ),
    )
    def sc_add(x_hbm, pos_hbm, out_hbm, pos_v, x_v, lsem0, lsem1, ssem0, ssem1):
        wid = lax.axis_index("s") * _NC + lax.axis_index("c")
        base = wid * rows_per_w
        lsems = (lsem0, lsem1)
        ssems = (ssem0, ssem1)

        def fire_loads(t, s):
            r0 = base + t * _P
            hs = [pltpu.async_copy(pos_hbm.at[pl.ds(r0, _P)], pos_v.at[s], lsems[s])]
            for b in range(batch):
                hs.append(
                    pltpu.async_copy(x_hbm.at[b, pl.ds(r0, _P)], x_v.at[s, b], lsems[s])
                )
            return hs

        def fire_stores(t, s):
            r0 = base + t * _P
            return [
                pltpu.async_copy(x_v.at[s, b], out_hbm.at[b, pl.ds(r0, _P)], ssems[s])
                for b in range(batch)
            ]

        def compute(s):
            def row_body(r, carry):
                for c in range(ncol):
                    pv = pos_v[s, r, pl.ds(c * _L, _L)]
                    for b in range(batch):
                        plsc.addupdate(x_v.at[s, b, r, pl.ds(c * _L, _L)], pv)
                return carry

            lax.fori_loop(0, _P, row_body, 0)

        pending_loads = {0: fire_loads(0, 0)}
        pending_stores = {}
        for t in range(steps):
            s = t & 1
            for h in pending_loads.pop(t):
                h.wait()
            if t + 1 < steps:
                if t - 1 in pending_stores:
                    for h in pending_stores.pop(t - 1):
                        h.wait()
                pending_loads[t + 1] = fire_loads(t + 1, s ^ 1)
            compute(s)
            pending_stores[t] = fire_stores(t, s)
        for ts in sorted(pending_stores):
            for h in pending_stores[ts]:
                h.wait()

    return sc_add(x, pos)
